# baseline (speedup 1.0000x reference)
import functools

import jax
import jax.numpy as jnp
from jax import lax
from jax.experimental import pallas as pl
from jax.experimental.pallas import tpu as pltpu
from jax.experimental.pallas import tpu_sc as plsc

N = 10000
N_PAD = 10240
D_IN = 128
NC, NS = 2, 16
NW = NC * NS
CHUNK = 128
CPW = 80
E_PAD = NW * CPW * CHUNK
ROWS_PER_TILE = N_PAD // NS

MESH = plsc.VectorSubcoreMesh(core_axis_name="c", subcore_axis_name="s",
                              num_cores=NC, num_subcores=NS)


def _deg_body(dst_hbm, out_hbm, idx_v, hist_v):
    wid = lax.axis_index("s") * NC + lax.axis_index("c")
    epw = CPW * CHUNK
    pltpu.sync_copy(dst_hbm.at[pl.ds(wid * epw, epw)], idx_v)
    zeros16 = jnp.zeros((16,), jnp.float32)
    ones16 = jnp.ones((16,), jnp.float32)

    def zero(i, _):
        hist_v[pl.ds(i * 16, 16)] = zeros16
        return 0
    lax.fori_loop(0, N_PAD // 16, zero, 0)

    def acc(i, _):
        idx = idx_v[pl.ds(i * 16, 16)]
        plsc.addupdate_scatter(hist_v, [idx], ones16)
        return 0
    lax.fori_loop(0, epw // 16, acc, 0)
    pltpu.sync_copy(hist_v, out_hbm.at[wid])


_sc_deg = functools.partial(
    pl.kernel,
    _deg_body,
    out_type=jax.ShapeDtypeStruct((NW, N_PAD), jnp.float32),
    mesh=MESH,
    scratch_types=[
        pltpu.VMEM((CPW * CHUNK,), jnp.int32),
        pltpu.VMEM((N_PAD,), jnp.float32),
    ],
    compiler_params=pltpu.CompilerParams(use_tc_tiling_on_sc=False,
                                         needs_layout_passes=False),
)()


ECH = 64
KPT = E_PAD // NS // ECH


def _edge_pass_kernel(d):
    def body(h_hbm, src_hbm, dst_hbm, out_hbm,
             src_v, dst_v, rows0, rows1, s_sh, sem0, sem1):
        c = lax.axis_index("c")
        s = lax.axis_index("s")
        rpt = ROWS_PER_TILE

        @pl.when(c == 0)
        def _():
            zero16 = jnp.zeros((16,), jnp.float32)

            def zrow(r, _):
                for q in range(d // 16):
                    rows0[r, pl.ds(q * 16, 16)] = zero16
                return 0
            lax.fori_loop(0, ECH, zrow, 0)
            for b in range(rpt // ECH):
                pltpu.sync_copy(rows0, s_sh.at[pl.ds(s * rpt + b * ECH, ECH)])

            base = s * KPT
            pltpu.sync_copy(src_hbm.at[pl.ds(base, KPT)], src_v)
            pltpu.sync_copy(dst_hbm.at[pl.ds(base, KPT)], dst_v)
            plsc.subcore_barrier()
            pltpu.async_copy(h_hbm.at[src_v.at[0]], rows0, sem0)

            def chunk2(jj, _):
                j0 = jj * 2
                pltpu.async_copy(h_hbm.at[src_v.at[j0 + 1]], rows1, sem1)
                pltpu.make_async_copy(h_hbm.at[src_v.at[j0]], rows0, sem0).wait()
                pltpu.sync_copy(rows0, s_sh.at[dst_v.at[j0]], add=True)

                @pl.when(j0 + 2 < KPT)
                def _():
                    pltpu.async_copy(h_hbm.at[src_v.at[j0 + 2]], rows0, sem0)
                pltpu.make_async_copy(h_hbm.at[src_v.at[j0 + 1]], rows1, sem1).wait()
                pltpu.sync_copy(rows1, s_sh.at[dst_v.at[j0 + 1]], add=True)
                return 0
            lax.fori_loop(0, KPT // 2, chunk2, 0)
            plsc.subcore_barrier()
            pltpu.sync_copy(s_sh.at[pl.ds(s * rpt, rpt)],
                            out_hbm.at[pl.ds(s * rpt, rpt)])

    return functools.partial(
        pl.kernel,
        body,
        out_type=jax.ShapeDtypeStruct((N_PAD, d), jnp.float32),
        mesh=MESH,
        scratch_types=[
            pltpu.VMEM((KPT, ECH), jnp.int32),
            pltpu.VMEM((KPT, ECH), jnp.int32),
            pltpu.VMEM((ECH, d), jnp.float32),
            pltpu.VMEM((ECH, d), jnp.float32),
            pltpu.VMEM_SHARED((N_PAD, d), jnp.float32),
            pltpu.SemaphoreType.DMA,
            pltpu.SemaphoreType.DMA,
        ],
        compiler_params=pltpu.CompilerParams(use_tc_tiling_on_sc=False),
    )()


_BLK = 512
_GRID = N_PAD // _BLK


def _dinv_of(deg_blk):
    deg = jnp.sum(deg_blk, axis=0) + 1.0
    return lax.rsqrt(deg)


def _tc_first_body(deg_ref, x_ref, w_ref, o_ref):
    dinv = _dinv_of(deg_ref[...])
    h = jnp.dot(x_ref[...], w_ref[...], preferred_element_type=jnp.float32,
                precision=lax.Precision.HIGHEST)
    o_ref[...] = h * dinv[:, None]


def _tc_mid_body(deg_ref, sa_ref, h_ref, b_ref, w_ref, o_ref):
    dinv = _dinv_of(deg_ref[...])
    agg = sa_ref[...] + h_ref[...]
    t = jax.nn.relu(dinv[:, None] * agg + b_ref[...])
    o_ref[...] = jnp.dot(t, w_ref[...], preferred_element_type=jnp.float32,
                         precision=lax.Precision.HIGHEST) * dinv[:, None]


def _tc_final_body(deg_ref, sa_ref, h_ref, b_ref, o_ref):
    dinv = _dinv_of(deg_ref[...])
    agg = sa_ref[...] + h_ref[...]
    o_ref[...] = dinv[:, None] * agg + b_ref[...]


def _deg_spec():
    return pl.BlockSpec((NW, _BLK), lambda i: (0, i))


def _tc_first(deg_parts, xp, w):
    d = w.shape[1]
    return pl.pallas_call(
        _tc_first_body,
        grid=(_GRID,),
        in_specs=[_deg_spec(),
                  pl.BlockSpec((_BLK, D_IN), lambda i: (i, 0)),
                  pl.BlockSpec((D_IN, d), lambda i: (0, 0))],
        out_specs=pl.BlockSpec((_BLK, d), lambda i: (i, 0)),
        out_shape=jax.ShapeDtypeStruct((N_PAD, d), jnp.float32),
    )(deg_parts, xp, w)


def _tc_mid(deg_parts, s_parts, hprev, b, w):
    din, dout = w.shape
    return pl.pallas_call(
        _tc_mid_body,
        grid=(_GRID,),
        in_specs=[_deg_spec(),
                  pl.BlockSpec((_BLK, din), lambda i: (i, 0)),
                  pl.BlockSpec((_BLK, din), lambda i: (i, 0)),
                  pl.BlockSpec((1, din), lambda i: (0, 0)),
                  pl.BlockSpec((din, dout), lambda i: (0, 0))],
        out_specs=pl.BlockSpec((_BLK, dout), lambda i: (i, 0)),
        out_shape=jax.ShapeDtypeStruct((N_PAD, dout), jnp.float32),
    )(deg_parts, s_parts, hprev, b, w)


def _tc_final(deg_parts, s_parts, hprev, b):
    d = hprev.shape[1]
    return pl.pallas_call(
        _tc_final_body,
        grid=(_GRID,),
        in_specs=[_deg_spec(),
                  pl.BlockSpec((_BLK, d), lambda i: (i, 0)),
                  pl.BlockSpec((_BLK, d), lambda i: (i, 0)),
                  pl.BlockSpec((1, d), lambda i: (0, 0))],
        out_specs=pl.BlockSpec((_BLK, d), lambda i: (i, 0)),
        out_shape=jax.ShapeDtypeStruct((N_PAD, d), jnp.float32),
    )(deg_parts, s_parts, hprev, b)


_D1, _D2, _D3 = 112, 80, 64


def _pad2(a, r, c):
    return jnp.pad(a, ((0, r - a.shape[0]), (0, c - a.shape[1])))


def kernel(x, edge_index, W1, b1, W2, b2, W3, b3):
    xp = _pad2(x, N_PAD, D_IN)
    w1 = _pad2(W1, D_IN, _D1)
    w2 = _pad2(W2, _D1, _D2)
    w3 = _pad2(W3, _D2, _D3)
    b1p = jnp.pad(b1, (0, _D1 - b1.shape[0]))[None, :]
    b2p = jnp.pad(b2, (0, _D2 - b2.shape[0]))[None, :]
    b3p = jnp.pad(b3, (0, _D3 - b3.shape[0]))[None, :]

    e = edge_index.shape[1]
    src = jnp.pad(edge_index[0], (0, E_PAD - e))
    dst = jnp.pad(edge_index[1], (0, E_PAD - e),
                  constant_values=N)
    src3 = src.reshape(E_PAD // ECH, ECH)
    dst3 = dst.reshape(E_PAD // ECH, ECH)
    dst_flat = dst.reshape(-1)

    deg_parts = _sc_deg(dst_flat)

    h1 = _tc_first(deg_parts, xp, w1)
    s1 = _edge_pass_kernel(_D1)(h1, src3, dst3)
    h2 = _tc_mid(deg_parts, s1, h1, b1p, w2)
    s2 = _edge_pass_kernel(_D2)(h2, src3, dst3)
    h3 = _tc_mid(deg_parts, s2, h2, b2p, w3)
    s3 = _edge_pass_kernel(_D3)(h3, src3, dst3)
    outp = _tc_final(deg_parts, s3, h3, b3p)
    return outp[:N, :50]


# speedup vs baseline: 1.0629x; 1.0629x over previous
import functools

import jax
import jax.numpy as jnp
from jax import lax
from jax.experimental import pallas as pl
from jax.experimental.pallas import tpu as pltpu
from jax.experimental.pallas import tpu_sc as plsc

N = 10000
N_PAD = 10240
D_IN = 128
NC, NS = 2, 16
NW = NC * NS
CHUNK = 128
CPW = 80
E_PAD = NW * CPW * CHUNK
ROWS_PER_TILE = N_PAD // NS

MESH = plsc.VectorSubcoreMesh(core_axis_name="c", subcore_axis_name="s",
                              num_cores=NC, num_subcores=NS)


def _deg_body(dst_hbm, out_hbm, idx_v, hist_v):
    wid = lax.axis_index("s") * NC + lax.axis_index("c")
    epw = CPW * CHUNK
    pltpu.sync_copy(dst_hbm.at[pl.ds(wid * epw, epw)], idx_v)
    zeros16 = jnp.zeros((16,), jnp.float32)
    ones16 = jnp.ones((16,), jnp.float32)

    def zero(i, _):
        hist_v[pl.ds(i * 16, 16)] = zeros16
        return 0
    lax.fori_loop(0, N_PAD // 16, zero, 0)

    def acc(i, _):
        idx = idx_v[pl.ds(i * 16, 16)]
        plsc.addupdate_scatter(hist_v, [idx], ones16)
        return 0
    lax.fori_loop(0, epw // 16, acc, 0)
    pltpu.sync_copy(hist_v, out_hbm.at[wid])


_sc_deg = functools.partial(
    pl.kernel,
    _deg_body,
    out_type=jax.ShapeDtypeStruct((NW, N_PAD), jnp.float32),
    mesh=MESH,
    scratch_types=[
        pltpu.VMEM((CPW * CHUNK,), jnp.int32),
        pltpu.VMEM((N_PAD,), jnp.float32),
    ],
    compiler_params=pltpu.CompilerParams(use_tc_tiling_on_sc=False,
                                         needs_layout_passes=False),
)()


ECH = 128
KPT = E_PAD // NS // ECH
IBLK = 40
NBLK = KPT // IBLK


def _edge_pass_kernel(d):
    def body(h_hbm, src_hbm, dst_hbm, out_hbm,
             src_v, dst_v, rows0, rows1, s_sh, sem0, sem1, sem_i):
        c = lax.axis_index("c")
        s = lax.axis_index("s")
        rpt = ROWS_PER_TILE

        @pl.when(c == 0)
        def _():
            zero16 = jnp.zeros((16,), jnp.float32)

            def zrow(r, _):
                for q in range(d // 16):
                    rows0[r, pl.ds(q * 16, 16)] = zero16
                return 0
            lax.fori_loop(0, ECH, zrow, 0)
            for b in range(rpt // ECH):
                pltpu.sync_copy(rows0, s_sh.at[pl.ds(s * rpt + b * ECH, ECH)])

            base = s * KPT
            pltpu.sync_copy(src_hbm.at[pl.ds(base, IBLK)],
                            src_v.at[pl.ds(0, IBLK)])
            pltpu.sync_copy(dst_hbm.at[pl.ds(base, IBLK)],
                            dst_v.at[pl.ds(0, IBLK)])
            plsc.subcore_barrier()

            def block(blk, _):
                sb = lax.rem(blk, 2) * IBLK
                nb = (1 - lax.rem(blk, 2)) * IBLK
                nxt = base + (blk + 1) * IBLK

                @pl.when(blk + 1 < NBLK)
                def _():
                    pltpu.async_copy(src_hbm.at[pl.ds(nxt, IBLK)],
                                     src_v.at[pl.ds(nb, IBLK)], sem_i)
                    pltpu.async_copy(dst_hbm.at[pl.ds(nxt, IBLK)],
                                     dst_v.at[pl.ds(nb, IBLK)], sem_i)
                pltpu.async_copy(h_hbm.at[src_v.at[sb]], rows0, sem0)

                def chunk2(jj, _):
                    j0 = sb + jj * 2
                    pltpu.async_copy(h_hbm.at[src_v.at[j0 + 1]], rows1, sem1)
                    pltpu.make_async_copy(h_hbm.at[src_v.at[j0]],
                                          rows0, sem0).wait()
                    pltpu.sync_copy(rows0, s_sh.at[dst_v.at[j0]], add=True)

                    @pl.when(jj * 2 + 2 < IBLK)
                    def _():
                        pltpu.async_copy(h_hbm.at[src_v.at[j0 + 2]],
                                         rows0, sem0)
                    pltpu.make_async_copy(h_hbm.at[src_v.at[j0 + 1]],
                                          rows1, sem1).wait()
                    pltpu.sync_copy(rows1, s_sh.at[dst_v.at[j0 + 1]], add=True)
                    return 0
                lax.fori_loop(0, IBLK // 2, chunk2, 0)

                @pl.when(blk + 1 < NBLK)
                def _():
                    pltpu.make_async_copy(src_hbm.at[pl.ds(nxt, IBLK)],
                                          src_v.at[pl.ds(nb, IBLK)],
                                          sem_i).wait()
                    pltpu.make_async_copy(dst_hbm.at[pl.ds(nxt, IBLK)],
                                          dst_v.at[pl.ds(nb, IBLK)],
                                          sem_i).wait()
                return 0
            lax.fori_loop(0, NBLK, block, 0)
            plsc.subcore_barrier()
            pltpu.sync_copy(s_sh.at[pl.ds(s * rpt, rpt)],
                            out_hbm.at[pl.ds(s * rpt, rpt)])

    return functools.partial(
        pl.kernel,
        body,
        out_type=jax.ShapeDtypeStruct((N_PAD, d), jnp.float32),
        mesh=MESH,
        scratch_types=[
            pltpu.VMEM((2 * IBLK, ECH), jnp.int32),
            pltpu.VMEM((2 * IBLK, ECH), jnp.int32),
            pltpu.VMEM((ECH, d), jnp.float32),
            pltpu.VMEM((ECH, d), jnp.float32),
            pltpu.VMEM_SHARED((N_PAD, d), jnp.float32),
            pltpu.SemaphoreType.DMA,
            pltpu.SemaphoreType.DMA,
            pltpu.SemaphoreType.DMA,
        ],
        compiler_params=pltpu.CompilerParams(use_tc_tiling_on_sc=False),
    )()


_BLK = 512
_GRID = N_PAD // _BLK


def _dinv_of(deg_blk):
    deg = jnp.sum(deg_blk, axis=0) + 1.0
    return lax.rsqrt(deg)


def _tc_first_body(deg_ref, x_ref, w_ref, o_ref):
    dinv = _dinv_of(deg_ref[...])
    h = jnp.dot(x_ref[...], w_ref[...], preferred_element_type=jnp.float32,
                precision=lax.Precision.HIGHEST)
    o_ref[...] = h * dinv[:, None]


def _tc_mid_body(deg_ref, sa_ref, h_ref, b_ref, w_ref, o_ref):
    dinv = _dinv_of(deg_ref[...])
    agg = sa_ref[...] + h_ref[...]
    t = jax.nn.relu(dinv[:, None] * agg + b_ref[...])
    o_ref[...] = jnp.dot(t, w_ref[...], preferred_element_type=jnp.float32,
                         precision=lax.Precision.HIGHEST) * dinv[:, None]


def _tc_final_body(deg_ref, sa_ref, h_ref, b_ref, o_ref):
    dinv = _dinv_of(deg_ref[...])
    agg = sa_ref[...] + h_ref[...]
    o_ref[...] = dinv[:, None] * agg + b_ref[...]


def _deg_spec():
    return pl.BlockSpec((NW, _BLK), lambda i: (0, i))


def _tc_first(deg_parts, xp, w):
    d = w.shape[1]
    return pl.pallas_call(
        _tc_first_body,
        grid=(_GRID,),
        in_specs=[_deg_spec(),
                  pl.BlockSpec((_BLK, D_IN), lambda i: (i, 0)),
                  pl.BlockSpec((D_IN, d), lambda i: (0, 0))],
        out_specs=pl.BlockSpec((_BLK, d), lambda i: (i, 0)),
        out_shape=jax.ShapeDtypeStruct((N_PAD, d), jnp.float32),
    )(deg_parts, xp, w)


def _tc_mid(deg_parts, s_parts, hprev, b, w):
    din, dout = w.shape
    return pl.pallas_call(
        _tc_mid_body,
        grid=(_GRID,),
        in_specs=[_deg_spec(),
                  pl.BlockSpec((_BLK, din), lambda i: (i, 0)),
                  pl.BlockSpec((_BLK, din), lambda i: (i, 0)),
                  pl.BlockSpec((1, din), lambda i: (0, 0)),
                  pl.BlockSpec((din, dout), lambda i: (0, 0))],
        out_specs=pl.BlockSpec((_BLK, dout), lambda i: (i, 0)),
        out_shape=jax.ShapeDtypeStruct((N_PAD, dout), jnp.float32),
    )(deg_parts, s_parts, hprev, b, w)


def _tc_final(deg_parts, s_parts, hprev, b):
    d = hprev.shape[1]
    return pl.pallas_call(
        _tc_final_body,
        grid=(_GRID,),
        in_specs=[_deg_spec(),
                  pl.BlockSpec((_BLK, d), lambda i: (i, 0)),
                  pl.BlockSpec((_BLK, d), lambda i: (i, 0)),
                  pl.BlockSpec((1, d), lambda i: (0, 0))],
        out_specs=pl.BlockSpec((_BLK, d), lambda i: (i, 0)),
        out_shape=jax.ShapeDtypeStruct((N_PAD, d), jnp.float32),
    )(deg_parts, s_parts, hprev, b)


_D1, _D2, _D3 = 112, 80, 64


def _pad2(a, r, c):
    return jnp.pad(a, ((0, r - a.shape[0]), (0, c - a.shape[1])))


def kernel(x, edge_index, W1, b1, W2, b2, W3, b3):
    xp = _pad2(x, N_PAD, D_IN)
    w1 = _pad2(W1, D_IN, _D1)
    w2 = _pad2(W2, _D1, _D2)
    w3 = _pad2(W3, _D2, _D3)
    b1p = jnp.pad(b1, (0, _D1 - b1.shape[0]))[None, :]
    b2p = jnp.pad(b2, (0, _D2 - b2.shape[0]))[None, :]
    b3p = jnp.pad(b3, (0, _D3 - b3.shape[0]))[None, :]

    e = edge_index.shape[1]
    src = jnp.pad(edge_index[0], (0, E_PAD - e))
    dst = jnp.pad(edge_index[1], (0, E_PAD - e),
                  constant_values=N)
    src3 = src.reshape(E_PAD // ECH, ECH)
    dst3 = dst.reshape(E_PAD // ECH, ECH)
    dst_flat = dst.reshape(-1)

    deg_parts = _sc_deg(dst_flat)

    h1 = _tc_first(deg_parts, xp, w1)
    s1 = _edge_pass_kernel(_D1)(h1, src3, dst3)
    h2 = _tc_mid(deg_parts, s1, h1, b1p, w2)
    s2 = _edge_pass_kernel(_D2)(h2, src3, dst3)
    h3 = _tc_mid(deg_parts, s2, h2, b2p, w3)
    s3 = _edge_pass_kernel(_D3)(h3, src3, dst3)
    outp = _tc_final(deg_parts, s3, h3, b3p)
    return outp[:N, :50]


# speedup vs baseline: 1.2986x; 1.2218x over previous
import functools

import jax
import jax.numpy as jnp
from jax import lax
from jax.experimental import pallas as pl
from jax.experimental.pallas import tpu as pltpu
from jax.experimental.pallas import tpu_sc as plsc

N = 10000
N_PAD = 10240
D_IN = 128
NC, NS = 2, 16
NW = NC * NS
CHUNK = 128
CPW = 80
E_PAD = NW * CPW * CHUNK
ROWS_PER_TILE = N_PAD // NS

MESH = plsc.VectorSubcoreMesh(core_axis_name="c", subcore_axis_name="s",
                              num_cores=NC, num_subcores=NS)


def _deg_body(dst_hbm, out_hbm, idx_v, hist_v):
    wid = lax.axis_index("s") * NC + lax.axis_index("c")
    epw = CPW * CHUNK
    pltpu.sync_copy(dst_hbm.at[pl.ds(wid * epw, epw)], idx_v)
    zeros16 = jnp.zeros((16,), jnp.float32)
    ones16 = jnp.ones((16,), jnp.float32)

    def zero(i, _):
        hist_v[pl.ds(i * 16, 16)] = zeros16
        return 0
    lax.fori_loop(0, N_PAD // 16, zero, 0)

    def acc(i, _):
        idx = idx_v[pl.ds(i * 16, 16)]
        plsc.addupdate_scatter(hist_v, [idx], ones16)
        return 0
    lax.fori_loop(0, epw // 16, acc, 0)
    pltpu.sync_copy(hist_v, out_hbm.at[wid])


_sc_deg = functools.partial(
    pl.kernel,
    _deg_body,
    out_type=jax.ShapeDtypeStruct((NW, N_PAD), jnp.float32),
    mesh=MESH,
    scratch_types=[
        pltpu.VMEM((CPW * CHUNK,), jnp.int32),
        pltpu.VMEM((N_PAD,), jnp.float32),
    ],
    compiler_params=pltpu.CompilerParams(use_tc_tiling_on_sc=False,
                                         needs_layout_passes=False),
)()


NCU = 2
TOT_CH = E_PAD // CHUNK // NS
_KSPLIT = {112: (118, 42), 80: (132, 28), 64: (118, 42)}


def _edge_pass_kernel(d):
    k0, k1 = _KSPLIT[d]
    kmax = max(k0, k1)

    def body(h_hbm, src_hbm, dst_hbm, out0_hbm, out1_hbm,
             src_v, dst_v, rows0, rows1, s_sh, sem0, sem1):
        c = lax.axis_index("c")
        s = lax.axis_index("s")
        rpt = ROWS_PER_TILE

        zero16 = jnp.zeros((16,), jnp.float32)

        def zrow(r, _):
            for q in range(d // 16):
                rows0[r, pl.ds(q * 16, 16)] = zero16
            return 0
        lax.fori_loop(0, CHUNK, zrow, 0)
        for b in range(rpt // CHUNK):
            pltpu.sync_copy(rows0, s_sh.at[pl.ds(s * rpt + b * CHUNK, CHUNK)])

        def run(k, base):
            pltpu.sync_copy(src_hbm.at[pl.ds(base, k)], src_v.at[pl.ds(0, k)])
            pltpu.sync_copy(dst_hbm.at[pl.ds(base, k)], dst_v.at[pl.ds(0, k)])
            plsc.subcore_barrier()
            pltpu.async_copy(h_hbm.at[src_v.at[0]], rows0, sem0)

            def chunk2(jj, _):
                j0 = jj * 2
                pltpu.async_copy(h_hbm.at[src_v.at[j0 + 1]], rows1, sem1)
                pltpu.make_async_copy(h_hbm.at[src_v.at[j0]], rows0, sem0).wait()
                pltpu.sync_copy(rows0, s_sh.at[dst_v.at[j0]], add=True)

                @pl.when(j0 + 2 < k)
                def _():
                    pltpu.async_copy(h_hbm.at[src_v.at[j0 + 2]], rows0, sem0)
                pltpu.make_async_copy(h_hbm.at[src_v.at[j0 + 1]], rows1, sem1).wait()
                pltpu.sync_copy(rows1, s_sh.at[dst_v.at[j0 + 1]], add=True)
                return 0
            lax.fori_loop(0, k // 2, chunk2, 0)

        @pl.when(c == 0)
        def _():
            run(k0, s * k0)

        @pl.when(c == 1)
        def _():
            run(k1, NS * k0 + s * k1)
        plsc.subcore_barrier()

        def writeout(out_hbm):
            nblk = rpt // CHUNK
            for b in range(nblk):
                buf, sem = (rows0, sem0) if b % 2 == 0 else (rows1, sem1)
                if b >= 2:
                    pltpu.make_async_copy(
                        buf, out_hbm.at[pl.ds(s * rpt + (b - 2) * CHUNK, CHUNK)],
                        sem).wait()
                pltpu.sync_copy(s_sh.at[pl.ds(s * rpt + b * CHUNK, CHUNK)], buf)
                pltpu.async_copy(
                    buf, out_hbm.at[pl.ds(s * rpt + b * CHUNK, CHUNK)], sem)
            for b in range(max(0, nblk - 2), nblk):
                buf, sem = (rows0, sem0) if b % 2 == 0 else (rows1, sem1)
                pltpu.make_async_copy(
                    buf, out_hbm.at[pl.ds(s * rpt + b * CHUNK, CHUNK)],
                    sem).wait()

        @pl.when(c == 0)
        def _():
            writeout(out0_hbm)

        @pl.when(c == 1)
        def _():
            writeout(out1_hbm)

    return functools.partial(
        pl.kernel,
        body,
        out_type=(jax.ShapeDtypeStruct((N_PAD, d), jnp.float32),
                  jax.ShapeDtypeStruct((N_PAD, d), jnp.float32)),
        mesh=MESH,
        scratch_types=[
            pltpu.VMEM((kmax, CHUNK), jnp.int32),
            pltpu.VMEM((kmax, CHUNK), jnp.int32),
            pltpu.VMEM((CHUNK, d), jnp.float32),
            pltpu.VMEM((CHUNK, d), jnp.float32),
            pltpu.VMEM_SHARED((N_PAD, d), jnp.float32),
            pltpu.SemaphoreType.DMA,
            pltpu.SemaphoreType.DMA,
        ],
        compiler_params=pltpu.CompilerParams(use_tc_tiling_on_sc=False),
    )()


_BLK = 512
_GRID = N_PAD // _BLK


def _dinv_of(deg_blk):
    deg = jnp.sum(deg_blk, axis=0) + 1.0
    return lax.rsqrt(deg)


def _tc_first_body(deg_ref, x_ref, w_ref, o_ref):
    dinv = _dinv_of(deg_ref[...])
    h = jnp.dot(x_ref[...], w_ref[...], preferred_element_type=jnp.float32,
                precision=lax.Precision.HIGHEST)
    o_ref[...] = h * dinv[:, None]


def _tc_mid_body(deg_ref, sa_ref, sb_ref, h_ref, b_ref, w_ref, o_ref):
    dinv = _dinv_of(deg_ref[...])
    agg = sa_ref[...] + sb_ref[...] + h_ref[...]
    t = jax.nn.relu(dinv[:, None] * agg + b_ref[...])
    o_ref[...] = jnp.dot(t, w_ref[...], preferred_element_type=jnp.float32,
                         precision=lax.Precision.HIGHEST) * dinv[:, None]


def _tc_final_body(deg_ref, sa_ref, sb_ref, h_ref, b_ref, o_ref):
    dinv = _dinv_of(deg_ref[...])
    agg = sa_ref[...] + sb_ref[...] + h_ref[...]
    o_ref[...] = dinv[:, None] * agg + b_ref[...]


def _deg_spec():
    return pl.BlockSpec((NW, _BLK), lambda i: (0, i))


def _tc_first(deg_parts, xp, w):
    d = w.shape[1]
    return pl.pallas_call(
        _tc_first_body,
        grid=(_GRID,),
        in_specs=[_deg_spec(),
                  pl.BlockSpec((_BLK, D_IN), lambda i: (i, 0)),
                  pl.BlockSpec((D_IN, d), lambda i: (0, 0))],
        out_specs=pl.BlockSpec((_BLK, d), lambda i: (i, 0)),
        out_shape=jax.ShapeDtypeStruct((N_PAD, d), jnp.float32),
    )(deg_parts, xp, w)


def _tc_mid(deg_parts, s_parts, hprev, b, w):
    din, dout = w.shape
    return pl.pallas_call(
        _tc_mid_body,
        grid=(_GRID,),
        in_specs=[_deg_spec(),
                  pl.BlockSpec((_BLK, din), lambda i: (i, 0)),
                  pl.BlockSpec((_BLK, din), lambda i: (i, 0)),
                  pl.BlockSpec((_BLK, din), lambda i: (i, 0)),
                  pl.BlockSpec((1, din), lambda i: (0, 0)),
                  pl.BlockSpec((din, dout), lambda i: (0, 0))],
        out_specs=pl.BlockSpec((_BLK, dout), lambda i: (i, 0)),
        out_shape=jax.ShapeDtypeStruct((N_PAD, dout), jnp.float32),
    )(deg_parts, s_parts[0], s_parts[1], hprev, b, w)


def _tc_final(deg_parts, s_parts, hprev, b):
    d = hprev.shape[1]
    return pl.pallas_call(
        _tc_final_body,
        grid=(_GRID,),
        in_specs=[_deg_spec(),
                  pl.BlockSpec((_BLK, d), lambda i: (i, 0)),
                  pl.BlockSpec((_BLK, d), lambda i: (i, 0)),
                  pl.BlockSpec((_BLK, d), lambda i: (i, 0)),
                  pl.BlockSpec((1, d), lambda i: (0, 0))],
        out_specs=pl.BlockSpec((_BLK, d), lambda i: (i, 0)),
        out_shape=jax.ShapeDtypeStruct((N_PAD, d), jnp.float32),
    )(deg_parts, s_parts[0], s_parts[1], hprev, b)


_D1, _D2, _D3 = 112, 80, 64


def _pad2(a, r, c):
    return jnp.pad(a, ((0, r - a.shape[0]), (0, c - a.shape[1])))


def kernel(x, edge_index, W1, b1, W2, b2, W3, b3):
    xp = _pad2(x, N_PAD, D_IN)
    w1 = _pad2(W1, D_IN, _D1)
    w2 = _pad2(W2, _D1, _D2)
    w3 = _pad2(W3, _D2, _D3)
    b1p = jnp.pad(b1, (0, _D1 - b1.shape[0]))[None, :]
    b2p = jnp.pad(b2, (0, _D2 - b2.shape[0]))[None, :]
    b3p = jnp.pad(b3, (0, _D3 - b3.shape[0]))[None, :]

    e = edge_index.shape[1]
    src = jnp.pad(edge_index[0], (0, E_PAD - e))
    dst = jnp.pad(edge_index[1], (0, E_PAD - e),
                  constant_values=N)
    src3 = src.reshape(NW * CPW, CHUNK)
    dst3 = dst.reshape(NW * CPW, CHUNK)
    dst_flat = dst.reshape(-1)

    deg_parts = _sc_deg(dst_flat)

    h1 = _tc_first(deg_parts, xp, w1)
    s1 = _edge_pass_kernel(_D1)(h1, src3, dst3)
    h2 = _tc_mid(deg_parts, s1, h1, b1p, w2)
    s2 = _edge_pass_kernel(_D2)(h2, src3, dst3)
    h3 = _tc_mid(deg_parts, s2, h2, b2p, w3)
    s3 = _edge_pass_kernel(_D3)(h3, src3, dst3)
    outp = _tc_final(deg_parts, s3, h3, b3p)
    return outp[:N, :50]
